# baseline (speedup 1.0000x reference)
import functools

import jax
import jax.numpy as jnp
from jax import lax
from jax.experimental import pallas as pl
from jax.experimental.pallas import tpu as pltpu
from jax.experimental.pallas import tpu_sc as plsc


def _make_gather(n_rows: int, d: int, chunk: int, nbuf: int):
    info = plsc.get_sparse_core_info()
    nw = info.num_cores * info.num_subcores
    assert n_rows % nw == 0
    per_w = n_rows // nw
    assert per_w % chunk == 0
    m = per_w // chunk

    mesh = plsc.VectorSubcoreMesh(core_axis_name="c", subcore_axis_name="s")

    @functools.partial(
        pl.kernel,
        out_type=jax.ShapeDtypeStruct((n_rows, d), jnp.float32),
        mesh=mesh,
        scratch_types=[
            pltpu.VMEM((per_w,), jnp.int32),
            pltpu.VMEM((nbuf, chunk, d), jnp.float32),
        ]
        + [pltpu.SemaphoreType.DMA] * (2 * nbuf),
        compiler_params=pltpu.CompilerParams(use_tc_tiling_on_sc=False),
    )
    def gather_kernel(idx_hbm, table_hbm, out_hbm, idx_v, rows_v, *sems):
        gsem = sems[:nbuf]
        ssem = sems[nbuf:]
        wid = lax.axis_index("s") * info.num_cores + lax.axis_index("c")
        base = wid * per_w

        pltpu.sync_copy(idx_hbm.at[pl.ds(base, per_w)], idx_v)

        def gather_copy(ci, b):
            off = pl.multiple_of(ci * chunk, 8)
            return pltpu.make_async_copy(
                table_hbm.at[idx_v.at[pl.ds(off, chunk)]], rows_v.at[b], gsem[b]
            )

        def store_copy(ci, b):
            off = pl.multiple_of(base + ci * chunk, 8)
            return pltpu.make_async_copy(
                rows_v.at[b], out_hbm.at[pl.ds(off, chunk)], ssem[b]
            )

        for b in range(nbuf - 1):
            gather_copy(b, b).start()

        def outer_body(o, carry):
            for b in range(nbuf):
                i = o * nbuf + b
                j = i + nbuf - 1
                bj = (b + nbuf - 1) % nbuf

                if b == 0:
                    @pl.when(o > 0)
                    def _():
                        store_copy(i - 1, bj).wait()
                else:
                    store_copy(i - 1, bj).wait()

                if b == 0:
                    gather_copy(j, bj).start()
                else:
                    @pl.when(j < m)
                    def _():
                        gather_copy(j, bj).start()

                gather_copy(i, b).wait()
                store_copy(i, b).start()
            return carry

        lax.fori_loop(0, m // nbuf, outer_body, 0)
        store_copy(m - 1, (m - 1) % nbuf).wait()

    return gather_kernel


def kernel(in_feat, table):
    b, h = in_feat.shape
    v, d = table.shape
    n = b * h
    idx = in_feat.reshape(n).astype(jnp.int32)
    out = _make_gather(n, d, chunk=640, nbuf=4)(idx, table)
    return out.reshape(b, h, d)


# speedup vs baseline: 1.6274x; 1.6274x over previous
import functools

import jax
import jax.numpy as jnp
from jax import lax
from jax.experimental import pallas as pl
from jax.experimental.pallas import tpu as pltpu
from jax.experimental.pallas import tpu_sc as plsc


def _make_gather(n_b: int, n_h: int, d: int, rows_per_chunk: int, nbuf: int):
    info = plsc.get_sparse_core_info()
    nw = info.num_cores * info.num_subcores
    assert n_b % nw == 0
    b_per_w = n_b // nw
    assert b_per_w % rows_per_chunk == 0
    m = b_per_w // rows_per_chunk
    chunk = rows_per_chunk * n_h

    mesh = plsc.VectorSubcoreMesh(core_axis_name="c", subcore_axis_name="s")

    @functools.partial(
        pl.kernel,
        out_type=jax.ShapeDtypeStruct((n_b, n_h, d), jnp.float32),
        mesh=mesh,
        scratch_types=[
            pltpu.VMEM((b_per_w * n_h,), jnp.int32),
            pltpu.VMEM((nbuf, chunk, d), jnp.float32),
        ]
        + [pltpu.SemaphoreType.DMA] * (2 * nbuf),
        compiler_params=pltpu.CompilerParams(use_tc_tiling_on_sc=False),
    )
    def gather_kernel(idx_hbm, table_hbm, out_hbm, idx_v, rows_v, *sems):
        gsem = sems[:nbuf]
        ssem = sems[nbuf:]
        wid = lax.axis_index("s") * info.num_cores + lax.axis_index("c")
        base_b = wid * b_per_w

        pltpu.sync_copy(idx_hbm.at[pl.ds(base_b * n_h, b_per_w * n_h)], idx_v)

        def gather_copy(ci, b):
            off = pl.multiple_of(ci * chunk, 8)
            return pltpu.make_async_copy(
                table_hbm.at[idx_v.at[pl.ds(off, chunk)]], rows_v.at[b], gsem[b]
            )

        def store_copies(ci, b):
            row0 = base_b + ci * rows_per_chunk
            return [
                pltpu.make_async_copy(
                    rows_v.at[b, pl.ds(r * n_h, n_h)],
                    out_hbm.at[row0 + r],
                    ssem[b],
                )
                for r in range(rows_per_chunk)
            ]

        for b in range(nbuf - 1):
            gather_copy(b, b).start()

        def outer_body(o, carry):
            for b in range(nbuf):
                i = o * nbuf + b
                j = i + nbuf - 1
                bj = (b + nbuf - 1) % nbuf

                if b == 0:
                    @pl.when(o > 0)
                    def _():
                        for c in store_copies(i - 1, bj):
                            c.wait()
                else:
                    for c in store_copies(i - 1, bj):
                        c.wait()

                if b == 0:
                    gather_copy(j, bj).start()
                else:
                    @pl.when(j < m)
                    def _():
                        gather_copy(j, bj).start()

                gather_copy(i, b).wait()
                for c in store_copies(i, b):
                    c.start()
            return carry

        lax.fori_loop(0, m // nbuf, outer_body, 0)
        for c in store_copies(m - 1, (m - 1) % nbuf):
            c.wait()

    return gather_kernel


def kernel(in_feat, table):
    b, h = in_feat.shape
    v, d = table.shape
    idx = in_feat.reshape(b * h).astype(jnp.int32)
    return _make_gather(b, h, d, rows_per_chunk=16, nbuf=4)(idx, table)
